# baseline (speedup 1.0000x reference)
import jax
import jax.numpy as jnp
from jax import lax
from jax.experimental import pallas as pl
from jax.experimental.pallas import tpu as pltpu
from jax.experimental.pallas import tpu_sc as plsc

N_NODES = 10000
N_EDGES = 320000
D_IN = 128
D_HID = 256

NC = 2
NS = 16
L = 16
C = 128
N_PAD = 10240
HN = 5120
ACC_ROWS = 5376
ZROWS = ACC_ROWS // NS
WROWS = HN // NS
DUMMY = HN
DST_PAD = 2 * HN

CE = 64
BC = 30
NB = 11
E_PAD2 = NS * NB * BC * CE


def _chunk_pad(a, e_pad, fill, lead_shape):
    pad = e_pad - N_EDGES
    ap = jnp.concatenate([a, jnp.full((pad,), fill, jnp.int32)])
    return ap.reshape(lead_shape)


def _phase_dst(dst):
    outs = []
    for p in range(2):
        lo = p * HN
        inr = (dst >= lo) & (dst < lo + HN)
        outs.append(jnp.where(inr, dst - lo, DUMMY))
    return jnp.stack(outs)


def _make_sc_agg2():
    dh = D_HID // 2
    mesh = plsc.VectorSubcoreMesh(core_axis_name="c", subcore_axis_name="s")
    out_types = (jax.ShapeDtypeStruct((2, N_PAD, dh), jnp.float32),)
    scratch = [
        pltpu.VMEM((BC, CE), jnp.int32),
        pltpu.VMEM((BC, CE), jnp.int32),
        pltpu.VMEM((CE, dh), jnp.float32),
        pltpu.VMEM((CE, dh), jnp.float32),
        pltpu.VMEM((CE, dh), jnp.float32),
        pltpu.VMEM_SHARED((ACC_ROWS, dh), jnp.float32),
        pltpu.SemaphoreType.DMA,
        pltpu.SemaphoreType.DMA,
        pltpu.SemaphoreType.DMA,
        pltpu.SemaphoreType.DMA,
        pltpu.SemaphoreType.DMA,
        pltpu.SemaphoreType.DMA,
    ]

    def body(h2d, srcs, dsts, zrow, out_agg, src_blk, dst_blk, r0, r1, r2,
             acc_sh, g0, g1, g2, v0, v1, v2):
        rows = (r0, r1, r2)
        gs = (g0, g1, g2)
        vs = (v0, v1, v2)
        c = lax.axis_index("c")
        s = lax.axis_index("s")

        def gather(j, rb, sem):
            pltpu.async_copy(h2d.at[src_blk.at[j]], rb, sem)

        def gather_wait(j, rb, sem):
            pltpu.make_async_copy(h2d.at[src_blk.at[j]], rb, sem).wait()

        def scat(j, rb, sem):
            pltpu.async_copy(rb, acc_sh.at[dst_blk.at[j]], sem, add=True)

        def scat_wait(j, rb, sem):
            pltpu.make_async_copy(rb, acc_sh.at[dst_blk.at[j]],
                                  sem).wait()

        for p in range(2):
            z0 = s * ZROWS
            pltpu.sync_copy(zrow, r0)
            for k in range(5):
                pltpu.sync_copy(r0, acc_sh.at[pl.ds(z0 + k * CE, CE)])
            pltpu.sync_copy(r0.at[pl.ds(0, 16)],
                            acc_sh.at[pl.ds(z0 + 5 * CE, 16)])
            plsc.subcore_barrier()

            def block(bi, _):
                pltpu.sync_copy(srcs.at[c, s * NB + bi], src_blk)
                pltpu.sync_copy(dsts.at[p, s * NB + bi], dst_blk)
                for b in range(3):
                    gather(b, rows[b], gs[b])

                def gen(kk, _):
                    j0 = 3 * kk
                    for b in range(3):
                        gather_wait(j0 + b, rows[b], gs[b])
                        scat(j0 + b, rows[b], vs[b])
                    for b in range(3):
                        scat_wait(j0 + b, rows[b], vs[b])
                        gather(j0 + 3 + b, rows[b], gs[b])
                    return 0
                lax.fori_loop(0, BC // 3 - 1, gen, 0)
                for b in range(3):
                    gather_wait(BC - 3 + b, rows[b], gs[b])
                    scat(BC - 3 + b, rows[b], vs[b])
                for b in range(3):
                    scat_wait(BC - 3 + b, rows[b], vs[b])
                return 0
            lax.fori_loop(0, NB, block, 0)
            plsc.subcore_barrier()
            w0 = s * WROWS
            for k in range(WROWS // CE):
                pltpu.sync_copy(acc_sh.at[pl.ds(w0 + k * CE, CE)], r0)
                pltpu.sync_copy(
                    r0, out_agg.at[c, pl.ds(p * HN + w0 + k * CE, CE)])

    return pl.kernel(body, out_type=out_types, mesh=mesh,
                     scratch_types=scratch)


def _make_tc_layer1():
    R = 1024
    nb = N_PAD // R
    row = lambda i: (i, 0)
    full = lambda i: (0, 0)
    in_specs = [
        pl.BlockSpec((R, D_IN), row),
        pl.BlockSpec((1, R, D_IN), lambda i: (0, i, 0)),
        pl.BlockSpec((1, R, D_IN), lambda i: (1, i, 0)),
        pl.BlockSpec((D_IN, D_HID), full),
        pl.BlockSpec((D_IN, D_HID), full),
        pl.BlockSpec((1, D_HID), full),
    ]

    def body(xb, ag, dg, ws, wn, b, o):
        invd = 1.0 / jnp.maximum(dg[0][:, 0:1], 1.0)
        mean = ag[0] * invd
        acc = jnp.dot(xb[...], ws[...], preferred_element_type=jnp.float32)
        acc += jnp.dot(mean, wn[...], preferred_element_type=jnp.float32)
        acc += b[...]
        acc = jnp.maximum(acc, 0.0)
        o[0] = acc[:, :128]
        o[1] = acc[:, 128:]

    return pl.pallas_call(
        body, grid=(nb,), in_specs=in_specs,
        out_specs=pl.BlockSpec((2, R, 128), lambda i: (0, i, 0)),
        out_shape=jax.ShapeDtypeStruct((2, N_PAD, 128), jnp.float32))


def _make_tc_layer23(relu, stacked_out):
    dh = D_HID // 2
    R = 1024
    nb = N_PAD // R
    row_l = lambda i: (i, 0)
    row_r = lambda i: (i + nb, 0)
    full = lambda i: (0, 0)
    in_specs = [
        pl.BlockSpec((R, dh), row_l),
        pl.BlockSpec((R, dh), row_r),
        pl.BlockSpec((R, dh), row_l),
        pl.BlockSpec((R, dh), row_r),
        pl.BlockSpec((1, R, D_IN), lambda i: (1, i, 0)),
        pl.BlockSpec((D_HID, D_HID), full),
        pl.BlockSpec((D_HID, D_HID), full),
        pl.BlockSpec((1, D_HID), full),
    ]
    if stacked_out:
        out_spec = pl.BlockSpec((2, R, 128), lambda i: (0, i, 0))
        out_shape = jax.ShapeDtypeStruct((2, N_PAD, 128), jnp.float32)
    else:
        out_spec = pl.BlockSpec((R, D_HID), row_l)
        out_shape = jax.ShapeDtypeStruct((N_PAD, D_HID), jnp.float32)

    def body(h_l, h_r, a_l, a_r, dg, ws, wn, b, o):
        invd = 1.0 / jnp.maximum(dg[0][:, 0:1], 1.0)
        acc = jnp.dot(h_l[...], ws[:dh, :], preferred_element_type=jnp.float32)
        acc += jnp.dot(h_r[...], ws[dh:, :], preferred_element_type=jnp.float32)
        acc += jnp.dot(a_l[...] * invd, wn[:dh, :],
                       preferred_element_type=jnp.float32)
        acc += jnp.dot(a_r[...] * invd, wn[dh:, :],
                       preferred_element_type=jnp.float32)
        acc += b[...]
        if relu:
            acc = jnp.maximum(acc, 0.0)
        if stacked_out:
            o[0] = acc[:, :128]
            o[1] = acc[:, 128:]
        else:
            o[...] = acc

    return pl.pallas_call(
        body, grid=(nb,), in_specs=in_specs, out_specs=out_spec,
        out_shape=out_shape)


_sc_agg2 = _make_sc_agg2()
_tc_l1 = _make_tc_layer1()
_tc_l2 = _make_tc_layer23(relu=True, stacked_out=True)
_tc_l3 = _make_tc_layer23(relu=False, stacked_out=False)


def kernel(x, edge_index, W_self1, W_neigh1, b1, W_self2, W_neigh2, b2,
           W_self3, W_neigh3, b3):
    src = edge_index[0].astype(jnp.int32)
    dst = edge_index[1].astype(jnp.int32)
    src2 = _chunk_pad(src, E_PAD2, 0, (NS * NB, BC, CE))
    srcs2 = jnp.stack([src2, src2 + N_PAD])
    dst2 = _chunk_pad(dst, E_PAD2, DST_PAD, (NS * NB, BC, CE))
    dsts2 = _phase_dst(dst2)
    svec = jnp.repeat(jnp.arange(NS, dtype=jnp.int32), NB)
    dsts2 = jnp.where(dsts2 == DUMMY,
                      DUMMY + svec[None, :, None, None], dsts2)

    zrow = jnp.zeros((CE, D_IN), jnp.float32)

    xp = jnp.pad(x, ((0, N_PAD - N_NODES), (0, 0)))
    onehot = jnp.zeros((N_PAD, D_IN), jnp.float32).at[:, 0].set(1.0)
    x_aug = jnp.concatenate([xp, onehot], axis=0)
    b1r = b1.reshape(1, D_HID)
    b2r = b2.reshape(1, D_HID)
    b3r = b3.reshape(1, D_HID)

    (agg1,) = _sc_agg2(x_aug, srcs2, dsts2, zrow)
    h2 = _tc_l1(xp, agg1, agg1, W_self1, W_neigh1, b1r)
    h2d = h2.reshape(2 * N_PAD, 128)

    (agg2,) = _sc_agg2(h2d, srcs2, dsts2, zrow)
    a2 = agg2.reshape(2 * N_PAD, 128)
    h3 = _tc_l2(h2d, h2d, a2, a2, agg1, W_self2, W_neigh2, b2r)
    h3d = h3.reshape(2 * N_PAD, 128)

    (agg3,) = _sc_agg2(h3d, srcs2, dsts2, zrow)
    a3 = agg3.reshape(2 * N_PAD, 128)
    out = _tc_l3(h3d, h3d, a3, a3, agg1, W_self3, W_neigh3, b3r)
    return out[:N_NODES]


# speedup vs baseline: 1.1755x; 1.1755x over previous
import jax
import jax.numpy as jnp
from jax import lax
from jax.experimental import pallas as pl
from jax.experimental.pallas import tpu as pltpu
from jax.experimental.pallas import tpu_sc as plsc

N_NODES = 10000
N_EDGES = 320000
D_IN = 128
D_HID = 256

NC = 2
NS = 16
L = 16
C = 128
N_PAD = 10240
HN = 5120
ACC_ROWS = 5376
ZROWS = ACC_ROWS // NS
WROWS = HN // NS
DUMMY = HN
DST_PAD = 2 * HN

CE = 96
BC = 24
NB = 9
E_PAD2 = NS * NB * BC * CE


def _chunk_pad(a, e_pad, fill, lead_shape):
    pad = e_pad - N_EDGES
    ap = jnp.concatenate([a, jnp.full((pad,), fill, jnp.int32)])
    return ap.reshape(lead_shape)


def _phase_dst(dst):
    outs = []
    for p in range(2):
        lo = p * HN
        inr = (dst >= lo) & (dst < lo + HN)
        outs.append(jnp.where(inr, dst - lo, DUMMY))
    return jnp.stack(outs)


def _make_sc_agg2():
    dh = D_HID // 2
    mesh = plsc.VectorSubcoreMesh(core_axis_name="c", subcore_axis_name="s")
    out_types = (jax.ShapeDtypeStruct((2, N_PAD, dh), jnp.float32),)
    scratch = [
        pltpu.VMEM((BC, CE), jnp.int32),
        pltpu.VMEM((BC, CE), jnp.int32),
        pltpu.VMEM((CE, dh), jnp.float32),
        pltpu.VMEM((CE, dh), jnp.float32),
        pltpu.VMEM_SHARED((ACC_ROWS, dh), jnp.float32),
        pltpu.SemaphoreType.DMA,
        pltpu.SemaphoreType.DMA,
        pltpu.SemaphoreType.DMA,
        pltpu.SemaphoreType.DMA,
    ]

    def body(h2d, srcs, dsts, zrow, out_agg, src_blk, dst_blk, r0, r1,
             acc_sh, g0, g1, v0, v1):
        c = lax.axis_index("c")
        s = lax.axis_index("s")

        def gather(j, rb, sem):
            pltpu.async_copy(h2d.at[src_blk.at[j]], rb, sem)

        def gather_wait(j, rb, sem):
            pltpu.make_async_copy(h2d.at[src_blk.at[j]], rb, sem).wait()

        def scat(j, rb, sem):
            pltpu.async_copy(rb, acc_sh.at[dst_blk.at[j]], sem, add=True)

        def scat_wait(j, rb, sem):
            pltpu.make_async_copy(rb, acc_sh.at[dst_blk.at[j]],
                                  sem).wait()

        for p in range(2):
            z0 = s * ZROWS
            pltpu.sync_copy(zrow, r0)
            for k in range(ZROWS // CE):
                pltpu.sync_copy(r0, acc_sh.at[pl.ds(z0 + k * CE, CE)])
            zr = ZROWS % CE
            if zr:
                pltpu.sync_copy(
                    r0.at[pl.ds(0, zr)],
                    acc_sh.at[pl.ds(z0 + (ZROWS // CE) * CE, zr)])
            plsc.subcore_barrier()

            def block(bi, _):
                pltpu.sync_copy(srcs.at[c, s * NB + bi], src_blk)
                pltpu.sync_copy(dsts.at[p, s * NB + bi], dst_blk)
                gather(0, r0, g0)
                gather(1, r1, g1)

                def gen(kk, _):
                    j0 = 2 * kk
                    gather_wait(j0, r0, g0)
                    scat(j0, r0, v0)
                    gather_wait(j0 + 1, r1, g1)
                    scat(j0 + 1, r1, v1)
                    scat_wait(j0, r0, v0)
                    gather(j0 + 2, r0, g0)
                    scat_wait(j0 + 1, r1, v1)
                    gather(j0 + 3, r1, g1)
                    return 0
                lax.fori_loop(0, BC // 2 - 1, gen, 0)
                gather_wait(BC - 2, r0, g0)
                scat(BC - 2, r0, v0)
                gather_wait(BC - 1, r1, g1)
                scat(BC - 1, r1, v1)
                scat_wait(BC - 2, r0, v0)
                scat_wait(BC - 1, r1, v1)
                return 0
            lax.fori_loop(0, NB, block, 0)
            plsc.subcore_barrier()
            w0 = s * WROWS
            for k in range(WROWS // CE):
                pltpu.sync_copy(acc_sh.at[pl.ds(w0 + k * CE, CE)], r0)
                pltpu.sync_copy(
                    r0, out_agg.at[c, pl.ds(p * HN + w0 + k * CE, CE)])
            wr = WROWS % CE
            if wr:
                wo = (WROWS // CE) * CE
                pltpu.sync_copy(acc_sh.at[pl.ds(w0 + wo, wr)],
                                r0.at[pl.ds(0, wr)])
                pltpu.sync_copy(
                    r0.at[pl.ds(0, wr)],
                    out_agg.at[c, pl.ds(p * HN + w0 + wo, wr)])

    return pl.kernel(body, out_type=out_types, mesh=mesh,
                     scratch_types=scratch)


def _make_tc_layer1():
    R = 1024
    nb = N_PAD // R
    row = lambda i: (i, 0)
    full = lambda i: (0, 0)
    in_specs = [
        pl.BlockSpec((R, D_IN), row),
        pl.BlockSpec((1, R, D_IN), lambda i: (0, i, 0)),
        pl.BlockSpec((1, R, D_IN), lambda i: (1, i, 0)),
        pl.BlockSpec((D_IN, D_HID), full),
        pl.BlockSpec((D_IN, D_HID), full),
        pl.BlockSpec((1, D_HID), full),
    ]

    def body(xb, ag, dg, ws, wn, b, o):
        invd = 1.0 / jnp.maximum(dg[0][:, 0:1], 1.0)
        mean = ag[0] * invd
        acc = jnp.dot(xb[...], ws[...], preferred_element_type=jnp.float32)
        acc += jnp.dot(mean, wn[...], preferred_element_type=jnp.float32)
        acc += b[...]
        acc = jnp.maximum(acc, 0.0)
        o[0] = acc[:, :128]
        o[1] = acc[:, 128:]

    return pl.pallas_call(
        body, grid=(nb,), in_specs=in_specs,
        out_specs=pl.BlockSpec((2, R, 128), lambda i: (0, i, 0)),
        out_shape=jax.ShapeDtypeStruct((2, N_PAD, 128), jnp.float32))


def _make_tc_layer23(relu, stacked_out):
    dh = D_HID // 2
    R = 1024
    nb = N_PAD // R
    row_l = lambda i: (i, 0)
    row_r = lambda i: (i + nb, 0)
    full = lambda i: (0, 0)
    in_specs = [
        pl.BlockSpec((R, dh), row_l),
        pl.BlockSpec((R, dh), row_r),
        pl.BlockSpec((R, dh), row_l),
        pl.BlockSpec((R, dh), row_r),
        pl.BlockSpec((1, R, D_IN), lambda i: (1, i, 0)),
        pl.BlockSpec((D_HID, D_HID), full),
        pl.BlockSpec((D_HID, D_HID), full),
        pl.BlockSpec((1, D_HID), full),
    ]
    if stacked_out:
        out_spec = pl.BlockSpec((2, R, 128), lambda i: (0, i, 0))
        out_shape = jax.ShapeDtypeStruct((2, N_PAD, 128), jnp.float32)
    else:
        out_spec = pl.BlockSpec((R, D_HID), row_l)
        out_shape = jax.ShapeDtypeStruct((N_PAD, D_HID), jnp.float32)

    def body(h_l, h_r, a_l, a_r, dg, ws, wn, b, o):
        invd = 1.0 / jnp.maximum(dg[0][:, 0:1], 1.0)
        acc = jnp.dot(h_l[...], ws[:dh, :], preferred_element_type=jnp.float32)
        acc += jnp.dot(h_r[...], ws[dh:, :], preferred_element_type=jnp.float32)
        acc += jnp.dot(a_l[...] * invd, wn[:dh, :],
                       preferred_element_type=jnp.float32)
        acc += jnp.dot(a_r[...] * invd, wn[dh:, :],
                       preferred_element_type=jnp.float32)
        acc += b[...]
        if relu:
            acc = jnp.maximum(acc, 0.0)
        if stacked_out:
            o[0] = acc[:, :128]
            o[1] = acc[:, 128:]
        else:
            o[...] = acc

    return pl.pallas_call(
        body, grid=(nb,), in_specs=in_specs, out_specs=out_spec,
        out_shape=out_shape)


_sc_agg2 = _make_sc_agg2()
_tc_l1 = _make_tc_layer1()
_tc_l2 = _make_tc_layer23(relu=True, stacked_out=True)
_tc_l3 = _make_tc_layer23(relu=False, stacked_out=False)


def kernel(x, edge_index, W_self1, W_neigh1, b1, W_self2, W_neigh2, b2,
           W_self3, W_neigh3, b3):
    src = edge_index[0].astype(jnp.int32)
    dst = edge_index[1].astype(jnp.int32)
    src2 = _chunk_pad(src, E_PAD2, 0, (NS * NB, BC, CE))
    srcs2 = jnp.stack([src2, src2 + N_PAD])
    dst2 = _chunk_pad(dst, E_PAD2, DST_PAD, (NS * NB, BC, CE))
    dsts2 = _phase_dst(dst2)
    svec = jnp.repeat(jnp.arange(NS, dtype=jnp.int32), NB)
    dsts2 = jnp.where(dsts2 == DUMMY,
                      DUMMY + svec[None, :, None, None], dsts2)

    zrow = jnp.zeros((CE, D_IN), jnp.float32)

    xp = jnp.pad(x, ((0, N_PAD - N_NODES), (0, 0)))
    onehot = jnp.zeros((N_PAD, D_IN), jnp.float32).at[:, 0].set(1.0)
    x_aug = jnp.concatenate([xp, onehot], axis=0)
    b1r = b1.reshape(1, D_HID)
    b2r = b2.reshape(1, D_HID)
    b3r = b3.reshape(1, D_HID)

    (agg1,) = _sc_agg2(x_aug, srcs2, dsts2, zrow)
    h2 = _tc_l1(xp, agg1, agg1, W_self1, W_neigh1, b1r)
    h2d = h2.reshape(2 * N_PAD, 128)

    (agg2,) = _sc_agg2(h2d, srcs2, dsts2, zrow)
    a2 = agg2.reshape(2 * N_PAD, 128)
    h3 = _tc_l2(h2d, h2d, a2, a2, agg1, W_self2, W_neigh2, b2r)
    h3d = h3.reshape(2 * N_PAD, 128)

    (agg3,) = _sc_agg2(h3d, srcs2, dsts2, zrow)
    a3 = agg3.reshape(2 * N_PAD, 128)
    out = _tc_l3(h3d, h3d, a3, a3, agg1, W_self3, W_neigh3, b3r)
    return out[:N_NODES]


# speedup vs baseline: 1.5763x; 1.3410x over previous
import jax
import jax.numpy as jnp
from jax import lax
from jax.experimental import pallas as pl
from jax.experimental.pallas import tpu as pltpu
from jax.experimental.pallas import tpu_sc as plsc

N_NODES = 10000
N_EDGES = 320000
D_IN = 128
D_HID = 256

NC = 2
NS = 16
L = 16
C = 128
N_PAD = 10240
HN = 5120
ACC_ROWS = 5376
ZROWS = ACC_ROWS // NS
WROWS = HN // NS
DUMMY = HN
DST_PAD = 2 * HN

CE = 64
BC = 32
NB = 10
E_PAD2 = NS * NB * BC * CE


def _chunk_pad(a, e_pad, fill, lead_shape):
    pad = e_pad - N_EDGES
    ap = jnp.concatenate([a, jnp.full((pad,), fill, jnp.int32)])
    return ap.reshape(lead_shape)


def _phase_dst(dst):
    outs = []
    for p in range(2):
        lo = p * HN
        inr = (dst >= lo) & (dst < lo + HN)
        outs.append(jnp.where(inr, dst - lo, DUMMY))
    return jnp.stack(outs)


def _make_sc_agg2():
    dh = D_HID // 2
    mesh = plsc.VectorSubcoreMesh(core_axis_name="c", subcore_axis_name="s")
    out_types = (jax.ShapeDtypeStruct((2, N_PAD, dh), jnp.float32),)
    scratch = [
        pltpu.VMEM((BC, CE), jnp.int32),
        pltpu.VMEM((BC, CE), jnp.int32),
        pltpu.VMEM((CE, dh), jnp.float32),
        pltpu.VMEM((CE, dh), jnp.float32),
        pltpu.VMEM_SHARED((ACC_ROWS, dh), jnp.float32),
        pltpu.SemaphoreType.DMA,
        pltpu.SemaphoreType.DMA,
        pltpu.SemaphoreType.DMA,
        pltpu.SemaphoreType.DMA,
    ]

    def body(h2d, srcs, dsts, zrow, out_agg, src_blk, dst_blk, r0, r1,
             acc_sh, g0, g1, v0, v1):
        c = lax.axis_index("c")
        s = lax.axis_index("s")

        def gather(j, rb, sem):
            pltpu.async_copy(h2d.at[src_blk.at[j]], rb, sem)

        def gather_wait(j, rb, sem):
            pltpu.make_async_copy(h2d.at[src_blk.at[j]], rb, sem).wait()

        def scat(j, rb, sem):
            pltpu.async_copy(rb, acc_sh.at[dst_blk.at[j]], sem, add=True)

        def scat_wait(j, rb, sem):
            pltpu.make_async_copy(rb, acc_sh.at[dst_blk.at[j]],
                                  sem).wait()

        for p in range(2):
            z0 = s * ZROWS
            pltpu.sync_copy(zrow, r0)
            for k in range(5):
                pltpu.sync_copy(r0, acc_sh.at[pl.ds(z0 + k * CE, CE)])
            pltpu.sync_copy(r0.at[pl.ds(0, 16)],
                            acc_sh.at[pl.ds(z0 + 5 * CE, 16)])
            plsc.subcore_barrier()

            def block(bi, _):
                pltpu.sync_copy(srcs.at[c, s * NB + bi], src_blk)
                pltpu.sync_copy(dsts.at[p, s * NB + bi], dst_blk)
                gather(0, r0, g0)
                gather(1, r1, g1)

                def gen(kk, _):
                    j0 = 2 * kk
                    gather_wait(j0, r0, g0)
                    scat(j0, r0, v0)
                    gather_wait(j0 + 1, r1, g1)
                    scat(j0 + 1, r1, v1)
                    scat_wait(j0, r0, v0)
                    gather(j0 + 2, r0, g0)
                    scat_wait(j0 + 1, r1, v1)
                    gather(j0 + 3, r1, g1)
                    return 0
                lax.fori_loop(0, BC // 2 - 1, gen, 0)
                gather_wait(BC - 2, r0, g0)
                scat(BC - 2, r0, v0)
                gather_wait(BC - 1, r1, g1)
                scat(BC - 1, r1, v1)
                scat_wait(BC - 2, r0, v0)
                scat_wait(BC - 1, r1, v1)
                return 0
            lax.fori_loop(0, NB, block, 0)
            plsc.subcore_barrier()
            w0 = s * WROWS
            for k in range(WROWS // CE):
                pltpu.sync_copy(acc_sh.at[pl.ds(w0 + k * CE, CE)], r0)
                pltpu.sync_copy(
                    r0, out_agg.at[c, pl.ds(p * HN + w0 + k * CE, CE)])

    return pl.kernel(body, out_type=out_types, mesh=mesh,
                     scratch_types=scratch)


def _make_tc_layer1():
    R = 1024
    nb = N_PAD // R
    row = lambda i: (i, 0)
    full = lambda i: (0, 0)
    in_specs = [
        pl.BlockSpec((R, D_IN), row),
        pl.BlockSpec((1, R, D_IN), lambda i: (0, i, 0)),
        pl.BlockSpec((1, R, D_IN), lambda i: (1, i, 0)),
        pl.BlockSpec((D_IN, D_HID), full),
        pl.BlockSpec((D_IN, D_HID), full),
        pl.BlockSpec((1, D_HID), full),
    ]

    def body(xb, ag, dg, ws, wn, b, o):
        invd = 1.0 / jnp.maximum(dg[0][:, 0:1], 1.0)
        mean = ag[0] * invd
        acc = jnp.dot(xb[...], ws[...], preferred_element_type=jnp.float32)
        acc += jnp.dot(mean, wn[...], preferred_element_type=jnp.float32)
        acc += b[...]
        acc = jnp.maximum(acc, 0.0)
        o[0] = acc[:, :128]
        o[1] = acc[:, 128:]

    return pl.pallas_call(
        body, grid=(nb,), in_specs=in_specs,
        out_specs=pl.BlockSpec((2, R, 128), lambda i: (0, i, 0)),
        out_shape=jax.ShapeDtypeStruct((2, N_PAD, 128), jnp.float32))


def _make_tc_layer23(relu, stacked_out):
    dh = D_HID // 2
    R = 1024
    nb = N_PAD // R
    row_l = lambda i: (i, 0)
    row_r = lambda i: (i + nb, 0)
    full = lambda i: (0, 0)
    in_specs = [
        pl.BlockSpec((R, dh), row_l),
        pl.BlockSpec((R, dh), row_r),
        pl.BlockSpec((R, dh), row_l),
        pl.BlockSpec((R, dh), row_r),
        pl.BlockSpec((1, R, D_IN), lambda i: (1, i, 0)),
        pl.BlockSpec((D_HID, D_HID), full),
        pl.BlockSpec((D_HID, D_HID), full),
        pl.BlockSpec((1, D_HID), full),
    ]
    if stacked_out:
        out_spec = pl.BlockSpec((2, R, 128), lambda i: (0, i, 0))
        out_shape = jax.ShapeDtypeStruct((2, N_PAD, 128), jnp.float32)
    else:
        out_spec = pl.BlockSpec((R, D_HID), row_l)
        out_shape = jax.ShapeDtypeStruct((N_PAD, D_HID), jnp.float32)

    def body(h_l, h_r, a_l, a_r, dg, ws, wn, b, o):
        invd = 1.0 / jnp.maximum(dg[0][:, 0:1], 1.0)
        acc = jnp.dot(h_l[...], ws[:dh, :], preferred_element_type=jnp.float32)
        acc += jnp.dot(h_r[...], ws[dh:, :], preferred_element_type=jnp.float32)
        acc += jnp.dot(a_l[...] * invd, wn[:dh, :],
                       preferred_element_type=jnp.float32)
        acc += jnp.dot(a_r[...] * invd, wn[dh:, :],
                       preferred_element_type=jnp.float32)
        acc += b[...]
        if relu:
            acc = jnp.maximum(acc, 0.0)
        if stacked_out:
            o[0] = acc[:, :128]
            o[1] = acc[:, 128:]
        else:
            o[...] = acc

    return pl.pallas_call(
        body, grid=(nb,), in_specs=in_specs, out_specs=out_spec,
        out_shape=out_shape)


_sc_agg2 = _make_sc_agg2()
_tc_l1 = _make_tc_layer1()
_tc_l2 = _make_tc_layer23(relu=True, stacked_out=True)
_tc_l3 = _make_tc_layer23(relu=False, stacked_out=False)


def kernel(x, edge_index, W_self1, W_neigh1, b1, W_self2, W_neigh2, b2,
           W_self3, W_neigh3, b3):
    src = edge_index[0].astype(jnp.int32)
    dst = edge_index[1].astype(jnp.int32)
    src2 = _chunk_pad(src, E_PAD2, 0, (NS * NB, BC, CE))
    srcs2 = jnp.stack([src2, src2 + N_PAD])
    dst2 = _chunk_pad(dst, E_PAD2, DST_PAD, (NS * NB, BC, CE))
    dsts2 = _phase_dst(dst2)
    svec = jnp.repeat(jnp.arange(NS, dtype=jnp.int32), NB)
    dsts2 = jnp.where(dsts2 == DUMMY,
                      DUMMY + svec[None, :, None, None], dsts2)

    zrow = jnp.zeros((CE, D_IN), jnp.float32)

    xp = jnp.pad(x, ((0, N_PAD - N_NODES), (0, 0)))
    onehot = jnp.zeros((N_PAD, D_IN), jnp.float32).at[:, 0].set(1.0)
    x_aug = jnp.concatenate([xp, onehot], axis=0)
    b1r = b1.reshape(1, D_HID)
    b2r = b2.reshape(1, D_HID)
    b3r = b3.reshape(1, D_HID)

    (agg1,) = _sc_agg2(x_aug, srcs2, dsts2, zrow)
    h2 = _tc_l1(xp, agg1, agg1, W_self1, W_neigh1, b1r)
    h2d = h2.reshape(2 * N_PAD, 128)

    (agg2,) = _sc_agg2(h2d, srcs2, dsts2, zrow)
    a2 = agg2.reshape(2 * N_PAD, 128)
    h3 = _tc_l2(h2d, h2d, a2, a2, agg1, W_self2, W_neigh2, b2r)
    h3d = h3.reshape(2 * N_PAD, 128)

    (agg3,) = _sc_agg2(h3d, srcs2, dsts2, zrow)
    a3 = agg3.reshape(2 * N_PAD, 128)
    out = _tc_l3(h3d, h3d, a3, a3, agg1, W_self3, W_neigh3, b3r)
    return out[:N_NODES]


# speedup vs baseline: 1.5764x; 1.0001x over previous
import jax
import jax.numpy as jnp
from jax import lax
from jax.experimental import pallas as pl
from jax.experimental.pallas import tpu as pltpu
from jax.experimental.pallas import tpu_sc as plsc

N_NODES = 10000
N_EDGES = 320000
D_IN = 128
D_HID = 256

NC = 2
NS = 16
L = 16
C = 128
N_PAD = 10240
HN = 5120
ACC_ROWS = 5376
ZROWS = ACC_ROWS // NS
WROWS = HN // NS
DUMMY = HN
DST_PAD = 2 * HN

CE = 64
BC = 64
NB = 5
E_PAD2 = NS * NB * BC * CE


def _chunk_pad(a, e_pad, fill, lead_shape):
    pad = e_pad - N_EDGES
    ap = jnp.concatenate([a, jnp.full((pad,), fill, jnp.int32)])
    return ap.reshape(lead_shape)


def _phase_dst(dst):
    outs = []
    for p in range(2):
        lo = p * HN
        inr = (dst >= lo) & (dst < lo + HN)
        outs.append(jnp.where(inr, dst - lo, DUMMY))
    return jnp.stack(outs)


def _make_sc_agg2():
    dh = D_HID // 2
    mesh = plsc.VectorSubcoreMesh(core_axis_name="c", subcore_axis_name="s")
    out_types = (jax.ShapeDtypeStruct((2, N_PAD, dh), jnp.float32),)
    scratch = [
        pltpu.VMEM((BC, CE), jnp.int32),
        pltpu.VMEM((BC, CE), jnp.int32),
        pltpu.VMEM((CE, dh), jnp.float32),
        pltpu.VMEM((CE, dh), jnp.float32),
        pltpu.VMEM_SHARED((ACC_ROWS, dh), jnp.float32),
        pltpu.SemaphoreType.DMA,
        pltpu.SemaphoreType.DMA,
        pltpu.SemaphoreType.DMA,
        pltpu.SemaphoreType.DMA,
    ]

    def body(h2d, srcs, dsts, zrow, out_agg, src_blk, dst_blk, r0, r1,
             acc_sh, g0, g1, v0, v1):
        c = lax.axis_index("c")
        s = lax.axis_index("s")

        def gather(j, rb, sem):
            pltpu.async_copy(h2d.at[src_blk.at[j]], rb, sem)

        def gather_wait(j, rb, sem):
            pltpu.make_async_copy(h2d.at[src_blk.at[j]], rb, sem).wait()

        def scat(j, rb, sem):
            pltpu.async_copy(rb, acc_sh.at[dst_blk.at[j]], sem, add=True)

        def scat_wait(j, rb, sem):
            pltpu.make_async_copy(rb, acc_sh.at[dst_blk.at[j]],
                                  sem).wait()

        for p in range(2):
            z0 = s * ZROWS
            pltpu.sync_copy(zrow, r0)
            for k in range(5):
                pltpu.sync_copy(r0, acc_sh.at[pl.ds(z0 + k * CE, CE)])
            pltpu.sync_copy(r0.at[pl.ds(0, 16)],
                            acc_sh.at[pl.ds(z0 + 5 * CE, 16)])
            plsc.subcore_barrier()

            def block(bi, _):
                pltpu.sync_copy(srcs.at[c, s * NB + bi], src_blk)
                pltpu.sync_copy(dsts.at[p, s * NB + bi], dst_blk)
                gather(0, r0, g0)
                gather(1, r1, g1)

                def gen(kk, _):
                    j0 = 2 * kk
                    gather_wait(j0, r0, g0)
                    scat(j0, r0, v0)
                    gather_wait(j0 + 1, r1, g1)
                    scat(j0 + 1, r1, v1)
                    scat_wait(j0, r0, v0)
                    gather(j0 + 2, r0, g0)
                    scat_wait(j0 + 1, r1, v1)
                    gather(j0 + 3, r1, g1)
                    return 0
                lax.fori_loop(0, BC // 2 - 1, gen, 0)
                gather_wait(BC - 2, r0, g0)
                scat(BC - 2, r0, v0)
                gather_wait(BC - 1, r1, g1)
                scat(BC - 1, r1, v1)
                scat_wait(BC - 2, r0, v0)
                scat_wait(BC - 1, r1, v1)
                return 0
            lax.fori_loop(0, NB, block, 0)
            plsc.subcore_barrier()
            w0 = s * WROWS
            for k in range(WROWS // CE):
                pltpu.sync_copy(acc_sh.at[pl.ds(w0 + k * CE, CE)], r0)
                pltpu.sync_copy(
                    r0, out_agg.at[c, pl.ds(p * HN + w0 + k * CE, CE)])

    return pl.kernel(body, out_type=out_types, mesh=mesh,
                     scratch_types=scratch)


def _make_tc_layer1():
    R = 1024
    nb = N_PAD // R
    row = lambda i: (i, 0)
    full = lambda i: (0, 0)
    in_specs = [
        pl.BlockSpec((R, D_IN), row),
        pl.BlockSpec((1, R, D_IN), lambda i: (0, i, 0)),
        pl.BlockSpec((1, R, D_IN), lambda i: (1, i, 0)),
        pl.BlockSpec((D_IN, D_HID), full),
        pl.BlockSpec((D_IN, D_HID), full),
        pl.BlockSpec((1, D_HID), full),
    ]

    def body(xb, ag, dg, ws, wn, b, o):
        invd = 1.0 / jnp.maximum(dg[0][:, 0:1], 1.0)
        mean = ag[0] * invd
        acc = jnp.dot(xb[...], ws[...], preferred_element_type=jnp.float32)
        acc += jnp.dot(mean, wn[...], preferred_element_type=jnp.float32)
        acc += b[...]
        acc = jnp.maximum(acc, 0.0)
        o[0] = acc[:, :128]
        o[1] = acc[:, 128:]

    return pl.pallas_call(
        body, grid=(nb,), in_specs=in_specs,
        out_specs=pl.BlockSpec((2, R, 128), lambda i: (0, i, 0)),
        out_shape=jax.ShapeDtypeStruct((2, N_PAD, 128), jnp.float32))


def _make_tc_layer23(relu, stacked_out):
    dh = D_HID // 2
    R = 1024
    nb = N_PAD // R
    row_l = lambda i: (i, 0)
    row_r = lambda i: (i + nb, 0)
    full = lambda i: (0, 0)
    in_specs = [
        pl.BlockSpec((R, dh), row_l),
        pl.BlockSpec((R, dh), row_r),
        pl.BlockSpec((R, dh), row_l),
        pl.BlockSpec((R, dh), row_r),
        pl.BlockSpec((1, R, D_IN), lambda i: (1, i, 0)),
        pl.BlockSpec((D_HID, D_HID), full),
        pl.BlockSpec((D_HID, D_HID), full),
        pl.BlockSpec((1, D_HID), full),
    ]
    if stacked_out:
        out_spec = pl.BlockSpec((2, R, 128), lambda i: (0, i, 0))
        out_shape = jax.ShapeDtypeStruct((2, N_PAD, 128), jnp.float32)
    else:
        out_spec = pl.BlockSpec((R, D_HID), row_l)
        out_shape = jax.ShapeDtypeStruct((N_PAD, D_HID), jnp.float32)

    def body(h_l, h_r, a_l, a_r, dg, ws, wn, b, o):
        invd = 1.0 / jnp.maximum(dg[0][:, 0:1], 1.0)
        acc = jnp.dot(h_l[...], ws[:dh, :], preferred_element_type=jnp.float32)
        acc += jnp.dot(h_r[...], ws[dh:, :], preferred_element_type=jnp.float32)
        acc += jnp.dot(a_l[...] * invd, wn[:dh, :],
                       preferred_element_type=jnp.float32)
        acc += jnp.dot(a_r[...] * invd, wn[dh:, :],
                       preferred_element_type=jnp.float32)
        acc += b[...]
        if relu:
            acc = jnp.maximum(acc, 0.0)
        if stacked_out:
            o[0] = acc[:, :128]
            o[1] = acc[:, 128:]
        else:
            o[...] = acc

    return pl.pallas_call(
        body, grid=(nb,), in_specs=in_specs, out_specs=out_spec,
        out_shape=out_shape)


_sc_agg2 = _make_sc_agg2()
_tc_l1 = _make_tc_layer1()
_tc_l2 = _make_tc_layer23(relu=True, stacked_out=True)
_tc_l3 = _make_tc_layer23(relu=False, stacked_out=False)


def kernel(x, edge_index, W_self1, W_neigh1, b1, W_self2, W_neigh2, b2,
           W_self3, W_neigh3, b3):
    src = edge_index[0].astype(jnp.int32)
    dst = edge_index[1].astype(jnp.int32)
    src2 = _chunk_pad(src, E_PAD2, 0, (NS * NB, BC, CE))
    srcs2 = jnp.stack([src2, src2 + N_PAD])
    dst2 = _chunk_pad(dst, E_PAD2, DST_PAD, (NS * NB, BC, CE))
    dsts2 = _phase_dst(dst2)
    svec = jnp.repeat(jnp.arange(NS, dtype=jnp.int32), NB)
    dsts2 = jnp.where(dsts2 == DUMMY,
                      DUMMY + svec[None, :, None, None], dsts2)

    zrow = jnp.zeros((CE, D_IN), jnp.float32)

    xp = jnp.pad(x, ((0, N_PAD - N_NODES), (0, 0)))
    onehot = jnp.zeros((N_PAD, D_IN), jnp.float32).at[:, 0].set(1.0)
    x_aug = jnp.concatenate([xp, onehot], axis=0)
    b1r = b1.reshape(1, D_HID)
    b2r = b2.reshape(1, D_HID)
    b3r = b3.reshape(1, D_HID)

    (agg1,) = _sc_agg2(x_aug, srcs2, dsts2, zrow)
    h2 = _tc_l1(xp, agg1, agg1, W_self1, W_neigh1, b1r)
    h2d = h2.reshape(2 * N_PAD, 128)

    (agg2,) = _sc_agg2(h2d, srcs2, dsts2, zrow)
    a2 = agg2.reshape(2 * N_PAD, 128)
    h3 = _tc_l2(h2d, h2d, a2, a2, agg1, W_self2, W_neigh2, b2r)
    h3d = h3.reshape(2 * N_PAD, 128)

    (agg3,) = _sc_agg2(h3d, srcs2, dsts2, zrow)
    a3 = agg3.reshape(2 * N_PAD, 128)
    out = _tc_l3(h3d, h3d, a3, a3, agg1, W_self3, W_neigh3, b3r)
    return out[:N_NODES]
